# baseline (speedup 1.0000x reference)
import jax
import jax.numpy as jnp
from jax import lax
from jax.experimental import pallas as pl
from jax.experimental.pallas import tpu as pltpu
from jax.experimental.pallas import tpu_sc as plsc

_N_NODES = 50000
_G = 256
_B = 16384
_K = 32
_HID = 256
_IN_DIM = 2 * _G

_NC, _NS = 2, 16
_NW = _NC * _NS
_IPW = _B // _NW
_NBUF = 4
_HK = _K // 2
_OC = 16
_NCH = _IPW // _OC
_VL = 16
_NCV = _IN_DIM // _VL


_RB = 1000


def _log_table_body(s_ref, u_ref, o_ref):
    cat = jnp.concatenate([jnp.log(s_ref[...] + 0.01),
                           jnp.log(u_ref[...] + 0.01)], axis=1)
    o_ref[...] = cat.reshape(4 * _RB, 128)


def _build_log_table(spliced, unspliced):
    return pl.pallas_call(
        _log_table_body,
        grid=(_N_NODES // _RB,),
        in_specs=[pl.BlockSpec((_RB, _G), lambda i: (i, 0)),
                  pl.BlockSpec((_RB, _G), lambda i: (i, 0))],
        out_specs=pl.BlockSpec((4 * _RB, 128), lambda i: (i, 0)),
        out_shape=jax.ShapeDtypeStruct((4 * _N_NODES, 128), jnp.float32),
    )(spliced, unspliced)


def _agg_body(lt128, idxh, wh, oh0, oh1, oh2, oh3, idx_v, w_v, idx4_v,
              oc0, oc1, oc2, oc3, oc4, oc5, oc6, oc7,
              buf0, buf1, buf2, buf3,
              sem0, sem1, sem2, sem3, fsem):
    ohs = (oh0, oh1, oh2, oh3)
    ocs = ((oc0, oc1, oc2, oc3), (oc4, oc5, oc6, oc7))
    bufs = (buf0, buf1, buf2, buf3)
    sems = (sem0, sem1, sem2, sem3)
    wid = lax.axis_index("s") * _NC + lax.axis_index("c")
    base = wid * _IPW
    pltpu.sync_copy(idxh.at[pl.ds(base, _IPW)], idx_v)
    pltpu.sync_copy(wh.at[pl.ds(base * _K, _IPW * _K)],
                    w_v.at[pl.ds(0, _IPW * _K)])

    def fire(i_local, b):
        iv_a = idx_v[i_local, pl.ds(0, _VL)] * 4
        iv_b = idx_v[i_local, pl.ds(_VL, _VL)] * 4
        for s in range(4):
            idx4_v[b, pl.ds(s * 2 * _VL, _VL)] = iv_a + s
            idx4_v[b, pl.ds(s * 2 * _VL + _VL, _VL)] = iv_b + s
        pltpu.make_async_copy(
            lt128.at[idx4_v.at[b]], bufs[b], sems[b]).start()

    def gather(i_local, b):
        return pltpu.make_async_copy(
            lt128.at[idx4_v.at[b]], bufs[b], sems[b])

    _LA = _NBUF

    for p in range(_LA):
        fire(p, p)

    def row_terms(buf, k, wk):
        return [wk * buf[(c // 8) * _K + k, pl.ds((c % 8) * _VL, _VL)]
                for c in range(_NCV)]

    def accum(i_local, row, buf, oc):
        wbase = i_local * _K
        wk0 = w_v[pl.ds(wbase, _VL)][0]
        accs = tuple(row_terms(buf, 0, wk0))

        def kbody(k, acc):
            wk = w_v[pl.ds(wbase + k, _VL)][0]
            return tuple(a + t for a, t in zip(acc, row_terms(buf, k, wk)))

        accs = lax.fori_loop(1, _K, kbody, accs)
        for c in range(_NCV):
            oc[c // 8][row, pl.ds((c % 8) * _VL, _VL)] = accs[c]

    def flush(par, cb):
        for q in range(4):
            pltpu.make_async_copy(
                ocs[par][q], ohs[q].at[pl.ds(base + cb, _OC)], fsem).start()

    def flush_wait(par, cb):
        for q in range(4):
            pltpu.make_async_copy(
                ocs[par][q], ohs[q].at[pl.ds(base + cb, _OC)], fsem).wait()

    for par in range(2):
        def chunk_body(ch2, carry, par=par):
            ch = 2 * ch2 + par
            cb = ch * _OC

            def grp_body(j, carry2):
                i0 = cb + _LA * j
                for p in range(_LA):
                    i = i0 + p
                    gather(i, p).wait()
                    accum(i, _LA * j + p, bufs[p], ocs[par])

                    @pl.when(i + _LA < _IPW)
                    def _(i=i, p=p):
                        fire(i + _LA, p)
                return carry2

            @pl.when(ch >= 2)
            def _():
                flush_wait(par, cb - 2 * _OC)

            lax.fori_loop(0, _OC // _LA, grp_body, 0)
            flush(par, cb)
            return carry

        if par == 0:
            loop0 = chunk_body
        else:
            loop1 = chunk_body

    def both(ch2, carry):
        carry = loop0(ch2, carry)
        carry = loop1(ch2, carry)
        return carry

    lax.fori_loop(0, _NCH // 2, both, 0)
    flush_wait(0, (_NCH - 2) * _OC)
    flush_wait(1, (_NCH - 1) * _OC)


def _aggregate(lt, idx, w_flat):
    f = pl.kernel(
        _agg_body,
        out_type=[jax.ShapeDtypeStruct((_B, 128), jnp.float32)] * 4,
        mesh=plsc.VectorSubcoreMesh(core_axis_name="c", subcore_axis_name="s",
                                    num_cores=_NC, num_subcores=_NS),
        scratch_types=[
            pltpu.VMEM((_IPW, _K), jnp.int32),
            pltpu.VMEM((_IPW * _K + _VL,), jnp.float32),
            pltpu.VMEM((_NBUF, 4 * _K), jnp.int32),
            pltpu.VMEM((_OC, 128), jnp.float32),
            pltpu.VMEM((_OC, 128), jnp.float32),
            pltpu.VMEM((_OC, 128), jnp.float32),
            pltpu.VMEM((_OC, 128), jnp.float32),
            pltpu.VMEM((_OC, 128), jnp.float32),
            pltpu.VMEM((_OC, 128), jnp.float32),
            pltpu.VMEM((_OC, 128), jnp.float32),
            pltpu.VMEM((_OC, 128), jnp.float32),
            pltpu.VMEM((4 * _K, 128), jnp.float32),
            pltpu.VMEM((4 * _K, 128), jnp.float32),
            pltpu.VMEM((4 * _K, 128), jnp.float32),
            pltpu.VMEM((4 * _K, 128), jnp.float32),
            pltpu.SemaphoreType.DMA,
            pltpu.SemaphoreType.DMA,
            pltpu.SemaphoreType.DMA,
            pltpu.SemaphoreType.DMA,
            pltpu.SemaphoreType.DMA,
        ],
        compiler_params=pltpu.CompilerParams(use_tc_tiling_on_sc=False))
    return f(lt, idx, w_flat)


def _mlp_body(enc_ref, a0_ref, a1_ref, a2_ref, a3_ref, w_ref,
              w1_ref, b1_ref, w2_ref, b2_ref, o_ref):
    s = jnp.sum(w_ref[...], axis=1, keepdims=True) + 1e-12
    x = jnp.concatenate(
        [a0_ref[...], a1_ref[...], a2_ref[...], a3_ref[...]], axis=1) / s
    h = lax.dot_general(x, w1_ref[...], (((1,), (1,)), ((), ())),
                        preferred_element_type=jnp.float32)
    h = jnp.maximum(h + b1_ref[...], 0.0)
    p = lax.dot_general(h, w2_ref[...], (((1,), (1,)), ((), ())),
                        preferred_element_type=jnp.float32)
    p = jnp.maximum(p + b2_ref[...], 0.0)
    o_ref[:, :_IN_DIM] = enc_ref[...]
    o_ref[:, _IN_DIM:] = p


def _mlp_concat(enc, agg, w, W1, b1, W2, b2):
    bm = 1024
    return pl.pallas_call(
        _mlp_body,
        grid=(_B // bm,),
        in_specs=[pl.BlockSpec((bm, _IN_DIM), lambda i: (i, 0)),
                  pl.BlockSpec((bm, 128), lambda i: (i, 0)),
                  pl.BlockSpec((bm, 128), lambda i: (i, 0)),
                  pl.BlockSpec((bm, 128), lambda i: (i, 0)),
                  pl.BlockSpec((bm, 128), lambda i: (i, 0)),
                  pl.BlockSpec((bm, _K), lambda i: (i, 0)),
                  pl.BlockSpec((_HID, _IN_DIM), lambda i: (0, 0)),
                  pl.BlockSpec((1, _HID), lambda i: (0, 0)),
                  pl.BlockSpec((_HID, _HID), lambda i: (0, 0)),
                  pl.BlockSpec((1, _HID), lambda i: (0, 0))],
        out_specs=pl.BlockSpec((bm, _IN_DIM + _HID), lambda i: (i, 0)),
        out_shape=jax.ShapeDtypeStruct((_B, _IN_DIM + _HID), jnp.float32),
    )(enc, *agg, w, W1, b1, W2, b2)


def kernel(encoder_input, neighbor_index, neighbor_weight, spliced_full,
           unspliced_full, W1, b1, W2, b2):
    idx = neighbor_index.astype(jnp.int32)
    lt = _build_log_table(spliced_full, unspliced_full)
    agg = _aggregate(lt, idx, neighbor_weight.reshape(-1))
    return _mlp_concat(encoder_input, agg, neighbor_weight,
                       W1, b1.reshape(1, _HID), W2, b2.reshape(1, _HID))


# speedup vs baseline: 1.0050x; 1.0050x over previous
import jax
import jax.numpy as jnp
from jax import lax
from jax.experimental import pallas as pl
from jax.experimental.pallas import tpu as pltpu
from jax.experimental.pallas import tpu_sc as plsc

_N_NODES = 50000
_G = 256
_B = 16384
_K = 32
_HID = 256
_IN_DIM = 2 * _G

_NC, _NS = 2, 16
_NW = _NC * _NS
_IPW = _B // _NW
_NBUF = 8
_HK = _K // 2
_OC = 16
_NCH = _IPW // _OC
_VL = 16
_NCV = _IN_DIM // _VL


_RB = 1000


def _log_table_body(s_ref, u_ref, o_ref):
    cat = jnp.concatenate([jnp.log(s_ref[...] + 0.01),
                           jnp.log(u_ref[...] + 0.01)], axis=1)
    o_ref[...] = cat.reshape(4 * _RB, 128)


def _build_log_table(spliced, unspliced):
    return pl.pallas_call(
        _log_table_body,
        grid=(_N_NODES // _RB,),
        in_specs=[pl.BlockSpec((_RB, _G), lambda i: (i, 0)),
                  pl.BlockSpec((_RB, _G), lambda i: (i, 0))],
        out_specs=pl.BlockSpec((4 * _RB, 128), lambda i: (i, 0)),
        out_shape=jax.ShapeDtypeStruct((4 * _N_NODES, 128), jnp.float32),
    )(spliced, unspliced)


def _agg_body(lt128, idxh, wh, oh0, oh1, oh2, oh3, idx_v, w_v, idx4_v,
              oc0, oc1, oc2, oc3, oc4, oc5, oc6, oc7,
              buf0, buf1, buf2, buf3, buf4, buf5, buf6, buf7,
              sem0, sem1, sem2, sem3, sem4, sem5, sem6, sem7, fsem):
    ohs = (oh0, oh1, oh2, oh3)
    ocs = ((oc0, oc1, oc2, oc3), (oc4, oc5, oc6, oc7))
    bufs = (buf0, buf1, buf2, buf3, buf4, buf5, buf6, buf7)
    sems = (sem0, sem1, sem2, sem3, sem4, sem5, sem6, sem7)
    wid = lax.axis_index("s") * _NC + lax.axis_index("c")
    base = wid * _IPW
    pltpu.sync_copy(idxh.at[pl.ds(base, _IPW)], idx_v)
    pltpu.sync_copy(wh.at[pl.ds(base * _K, _IPW * _K)],
                    w_v.at[pl.ds(0, _IPW * _K)])

    def fire(i_local, hb, b):
        iv4 = idx_v[i_local, pl.ds(hb * _HK, _VL)] * 4
        for s in range(4):
            idx4_v[b, pl.ds(s * _VL, _VL)] = iv4 + s
        pltpu.make_async_copy(
            lt128.at[idx4_v.at[b]], bufs[b], sems[b]).start()

    def gather(i_local, hb, b):
        return pltpu.make_async_copy(
            lt128.at[idx4_v.at[b]], bufs[b], sems[b])

    _LA = _NBUF // 2

    for p in range(_LA):
        for hb in range(2):
            fire(p, hb, p * 2 + hb)

    def row_terms(buf, k, wk):
        return [wk * buf[(c // 8) * _HK + k, pl.ds((c % 8) * _VL, _VL)]
                for c in range(_NCV)]

    def accum(i_local, row, buf_a, buf_b, oc):
        wbase = i_local * _K
        wk0 = w_v[pl.ds(wbase, _VL)][0]
        accs = tuple(row_terms(buf_a, 0, wk0))

        def kbody_a(k, acc):
            wk = w_v[pl.ds(wbase + k, _VL)][0]
            return tuple(a + t for a, t in zip(acc, row_terms(buf_a, k, wk)))

        accs = lax.fori_loop(1, _HK, kbody_a, accs)

        def kbody_b(k, acc):
            wk = w_v[pl.ds(wbase + _HK + k, _VL)][0]
            return tuple(a + t for a, t in zip(acc, row_terms(buf_b, k, wk)))

        accs = lax.fori_loop(0, _HK, kbody_b, accs)
        for c in range(_NCV):
            oc[c // 8][row, pl.ds((c % 8) * _VL, _VL)] = accs[c]

    def flush(par, cb):
        for q in range(4):
            pltpu.make_async_copy(
                ocs[par][q], ohs[q].at[pl.ds(base + cb, _OC)], fsem).start()

    def flush_wait(par, cb):
        for q in range(4):
            pltpu.make_async_copy(
                ocs[par][q], ohs[q].at[pl.ds(base + cb, _OC)], fsem).wait()

    for par in range(2):
        def chunk_body(ch2, carry, par=par):
            ch = 2 * ch2 + par
            cb = ch * _OC

            def grp_body(j, carry2):
                i0 = cb + _LA * j
                for p in range(_LA):
                    i = i0 + p
                    gather(i, 0, 2 * p).wait()
                    gather(i, 1, 2 * p + 1).wait()
                    accum(i, _LA * j + p, bufs[2 * p], bufs[2 * p + 1],
                          ocs[par])
                    for hb in range(2):
                        @pl.when(i + _LA < _IPW)
                        def _(i=i, hb=hb, p=p):
                            fire(i + _LA, hb, 2 * p + hb)
                return carry2

            @pl.when(ch >= 2)
            def _():
                flush_wait(par, cb - 2 * _OC)

            lax.fori_loop(0, _OC // _LA, grp_body, 0)
            flush(par, cb)
            return carry

        if par == 0:
            loop0 = chunk_body
        else:
            loop1 = chunk_body

    def both(ch2, carry):
        carry = loop0(ch2, carry)
        carry = loop1(ch2, carry)
        return carry

    lax.fori_loop(0, _NCH // 2, both, 0)
    flush_wait(0, (_NCH - 2) * _OC)
    flush_wait(1, (_NCH - 1) * _OC)


def _aggregate(lt, idx, w_flat):
    f = pl.kernel(
        _agg_body,
        out_type=[jax.ShapeDtypeStruct((_B, 128), jnp.float32)] * 4,
        mesh=plsc.VectorSubcoreMesh(core_axis_name="c", subcore_axis_name="s",
                                    num_cores=_NC, num_subcores=_NS),
        scratch_types=[
            pltpu.VMEM((_IPW, _K), jnp.int32),
            pltpu.VMEM((_IPW * _K + _VL,), jnp.float32),
            pltpu.VMEM((_NBUF, 4 * _HK), jnp.int32),
            pltpu.VMEM((_OC, 128), jnp.float32),
            pltpu.VMEM((_OC, 128), jnp.float32),
            pltpu.VMEM((_OC, 128), jnp.float32),
            pltpu.VMEM((_OC, 128), jnp.float32),
            pltpu.VMEM((_OC, 128), jnp.float32),
            pltpu.VMEM((_OC, 128), jnp.float32),
            pltpu.VMEM((_OC, 128), jnp.float32),
            pltpu.VMEM((_OC, 128), jnp.float32),
            pltpu.VMEM((4 * _HK, 128), jnp.float32),
            pltpu.VMEM((4 * _HK, 128), jnp.float32),
            pltpu.VMEM((4 * _HK, 128), jnp.float32),
            pltpu.VMEM((4 * _HK, 128), jnp.float32),
            pltpu.VMEM((4 * _HK, 128), jnp.float32),
            pltpu.VMEM((4 * _HK, 128), jnp.float32),
            pltpu.VMEM((4 * _HK, 128), jnp.float32),
            pltpu.VMEM((4 * _HK, 128), jnp.float32),
            pltpu.SemaphoreType.DMA,
            pltpu.SemaphoreType.DMA,
            pltpu.SemaphoreType.DMA,
            pltpu.SemaphoreType.DMA,
            pltpu.SemaphoreType.DMA,
            pltpu.SemaphoreType.DMA,
            pltpu.SemaphoreType.DMA,
            pltpu.SemaphoreType.DMA,
            pltpu.SemaphoreType.DMA,
        ],
        compiler_params=pltpu.CompilerParams(use_tc_tiling_on_sc=False))
    return f(lt, idx, w_flat)


def _mlp_body(enc_ref, a0_ref, a1_ref, a2_ref, a3_ref, w_ref,
              w1_ref, b1_ref, w2_ref, b2_ref, o_ref):
    s = jnp.sum(w_ref[...], axis=1, keepdims=True) + 1e-12
    x = jnp.concatenate(
        [a0_ref[...], a1_ref[...], a2_ref[...], a3_ref[...]], axis=1) / s
    h = lax.dot_general(x, w1_ref[...], (((1,), (1,)), ((), ())),
                        preferred_element_type=jnp.float32)
    h = jnp.maximum(h + b1_ref[...], 0.0)
    p = lax.dot_general(h, w2_ref[...], (((1,), (1,)), ((), ())),
                        preferred_element_type=jnp.float32)
    p = jnp.maximum(p + b2_ref[...], 0.0)
    o_ref[:, :_IN_DIM] = enc_ref[...]
    o_ref[:, _IN_DIM:] = p


def _mlp_concat(enc, agg, w, W1, b1, W2, b2):
    bm = 1024
    return pl.pallas_call(
        _mlp_body,
        grid=(_B // bm,),
        in_specs=[pl.BlockSpec((bm, _IN_DIM), lambda i: (i, 0)),
                  pl.BlockSpec((bm, 128), lambda i: (i, 0)),
                  pl.BlockSpec((bm, 128), lambda i: (i, 0)),
                  pl.BlockSpec((bm, 128), lambda i: (i, 0)),
                  pl.BlockSpec((bm, 128), lambda i: (i, 0)),
                  pl.BlockSpec((bm, _K), lambda i: (i, 0)),
                  pl.BlockSpec((_HID, _IN_DIM), lambda i: (0, 0)),
                  pl.BlockSpec((1, _HID), lambda i: (0, 0)),
                  pl.BlockSpec((_HID, _HID), lambda i: (0, 0)),
                  pl.BlockSpec((1, _HID), lambda i: (0, 0))],
        out_specs=pl.BlockSpec((bm, _IN_DIM + _HID), lambda i: (i, 0)),
        out_shape=jax.ShapeDtypeStruct((_B, _IN_DIM + _HID), jnp.float32),
    )(enc, *agg, w, W1, b1, W2, b2)


def kernel(encoder_input, neighbor_index, neighbor_weight, spliced_full,
           unspliced_full, W1, b1, W2, b2):
    idx = neighbor_index.astype(jnp.int32)
    lt = _build_log_table(spliced_full, unspliced_full)
    agg = _aggregate(lt, idx, neighbor_weight.reshape(-1))
    return _mlp_concat(encoder_input, agg, neighbor_weight,
                       W1, b1.reshape(1, _HID), W2, b2.reshape(1, _HID))
